# baseline (speedup 1.0000x reference)
import functools

import jax
import jax.numpy as jnp
from jax import lax
from jax.experimental import pallas as pl
from jax.experimental.pallas import tpu as pltpu
from jax.experimental.pallas import tpu_sc as plsc

NUM_FIELDS = 26
VOCAB = 1000000
BATCH = 16384

NC = 2
NS = 16
L = 16
NW = NC * NS
BPW = BATCH // NW
NCOL = BPW // L

CHW = 31232
TAIL = VOCAB - NW * CHW


def _body(x_ref, tab_ref, par_ref, out_ref, flat_ref,
          f32a, tbuf, xv2, parv,
          semi, semo, sem, semb, *xts):
    wid = lax.axis_index("s") * NC + lax.axis_index("c")
    base_b = wid * BPW
    src_off = pl.multiple_of(wid * CHW, 128)
    bufs = [f32a.at[pl.ds(0, CHW)], f32a.at[pl.ds(CHW, CHW)]]

    def cin(r):
        return pltpu.async_copy(
            tab_ref.at[r, pl.ds(src_off, CHW)], bufs[r % 2], semi
        )

    def cout(r):
        dst_off = pl.multiple_of(r * VOCAB + wid * CHW, 8)
        return pltpu.async_copy(
            bufs[r % 2], flat_ref.at[pl.ds(dst_off, CHW)], semo
        )

    ins = [cin(0)]
    outs = []
    for r in range(NUM_FIELDS):
        ins[r].wait()
        if r >= 1:
            outs[r - 1].wait()
        if r + 1 < NUM_FIELDS:
            ins.append(cin(r + 1))
        outs.append(cout(r))

    pltpu.sync_copy(par_ref, parv)
    iota = lax.iota(jnp.int32, L)
    HB = BPW // 2

    for h in range(2):
        pltpu.sync_copy(x_ref.at[pl.ds(base_b + h * HB, HB), :], xv2)
        for i in range(NUM_FIELDS):
            col = jnp.full((L,), i, jnp.int32)

            def trans(j, _, i=i, col=col, h=h):
                rows = j * L + iota
                xts[i][pl.ds(h * HB + j * L, L)] = (
                    plsc.load_gather(xv2, [rows, col]) + i * VOCAB
                )
                return 0

            lax.fori_loop(0, HB // L, trans, 0)

    outs[NUM_FIELDS - 1].wait()

    for r in range(NUM_FIELDS):

        @pl.when(wid == r)
        def _():
            pltpu.sync_copy(tab_ref.at[r, pl.ds(NW * CHW, TAIL)], tbuf)
            pltpu.sync_copy(tbuf, flat_ref.at[pl.ds(r * VOCAB + NW * CHW, TAIL)])

    plsc.subcore_barrier()
    pltpu.core_barrier(semb, core_axis_name="c")

    copies = []
    for i in range(NUM_FIELDS):
        copies.append(
            pltpu.async_copy(
                flat_ref.at[xts[i]], f32a.at[pl.ds(i * BPW, BPW)], sem
            )
        )
    for c in copies:
        c.wait()

    wvs = [
        plsc.load_gather(parv, [jnp.full((L,), i, jnp.int32)])
        for i in range(NUM_FIELDS)
    ]
    bconst = plsc.load_gather(parv, [jnp.full((L,), NUM_FIELDS, jnp.int32)])

    OV = NUM_FIELDS * BPW

    def colfn(j, _):
        b0 = j * L
        acc = bconst
        for i in range(NUM_FIELDS):
            acc = acc + wvs[i] * f32a[pl.ds(i * BPW + b0, L)]
        f32a[pl.ds(OV + b0, L)] = 1.0 / (1.0 + jnp.exp(-acc))
        return 0

    lax.fori_loop(0, NCOL, colfn, 0)

    pltpu.sync_copy(f32a.at[pl.ds(OV, BPW)], out_ref.at[pl.ds(base_b, BPW)])


@jax.jit
def _run(x, tables, W, b, bias):
    params = jnp.concatenate(
        [
            W.reshape(-1).astype(jnp.float32),
            (b + bias).reshape(-1).astype(jnp.float32),
            jnp.zeros(128 - NUM_FIELDS - 1, jnp.float32),
        ]
    )
    x = x.astype(jnp.int32)
    mesh = plsc.VectorSubcoreMesh(core_axis_name="c", subcore_axis_name="s")

    kern = pl.kernel(
        _body,
        out_type=(
            jax.ShapeDtypeStruct((BATCH,), jnp.float32),
            jax.ShapeDtypeStruct((NUM_FIELDS * VOCAB,), jnp.float32),
        ),
        mesh=mesh,
        compiler_params=pltpu.CompilerParams(needs_layout_passes=False),
        scratch_types=[
            pltpu.VMEM((2 * CHW,), jnp.float32),
            pltpu.VMEM((TAIL,), jnp.float32),
            pltpu.VMEM((BPW // 2, NUM_FIELDS), jnp.int32),
            pltpu.VMEM((128,), jnp.float32),
            pltpu.SemaphoreType.DMA,
            pltpu.SemaphoreType.DMA,
            pltpu.SemaphoreType.DMA,
            pltpu.SemaphoreType.BARRIER,
        ]
        + [pltpu.VMEM((BPW,), jnp.int32) for _ in range(NUM_FIELDS)],
    )
    out, _ = kern(x, tables, params)
    return out.reshape(BATCH, 1)


def kernel(x, tables, W, b, bias):
    return _run(x, tables, W, b, bias)


# speedup vs baseline: 1.1065x; 1.1065x over previous
import functools

import jax
import jax.numpy as jnp
from jax import lax
from jax.experimental import pallas as pl
from jax.experimental.pallas import tpu as pltpu
from jax.experimental.pallas import tpu_sc as plsc

NUM_FIELDS = 26
VOCAB = 1000000
BATCH = 16384

NC = 2
NS = 16
L = 16
NW = NC * NS
BPW = BATCH // NW
NCOL = BPW // L

CHW = 31232
TAIL = VOCAB - NW * CHW


def _stage_body(tab_ref, flat_ref, buf0, buf1, tbuf, semi, semo):
    wid = lax.axis_index("s") * NC + lax.axis_index("c")
    src_off = pl.multiple_of(wid * CHW, 128)
    bufs = [buf0, buf1]

    def cin(r):
        return pltpu.async_copy(
            tab_ref.at[r, pl.ds(src_off, CHW)], bufs[r % 2], semi
        )

    def cout(r):
        dst_off = pl.multiple_of(r * VOCAB + wid * CHW, 8)
        return pltpu.async_copy(
            bufs[r % 2], flat_ref.at[pl.ds(dst_off, CHW)], semo
        )

    ins = [cin(0)]
    outs = []
    for r in range(NUM_FIELDS):
        ins[r].wait()
        if r >= 1:
            outs[r - 1].wait()
        if r + 1 < NUM_FIELDS:
            ins.append(cin(r + 1))
        outs.append(cout(r))
    outs[NUM_FIELDS - 1].wait()

    for r in range(NUM_FIELDS):

        @pl.when(wid == r)
        def _():
            pltpu.sync_copy(tab_ref.at[r, pl.ds(NW * CHW, TAIL)], tbuf)
            pltpu.sync_copy(tbuf, flat_ref.at[pl.ds(r * VOCAB + NW * CHW, TAIL)])


def _gather_body(x_ref, tab_ref, par_ref, out_ref, xv2, rowsv, outv, parv,
                 sem, *xts):
    wid = lax.axis_index("s") * NC + lax.axis_index("c")
    base_b = wid * BPW

    pltpu.sync_copy(x_ref.at[pl.ds(base_b, BPW), :], xv2)
    pltpu.sync_copy(par_ref, parv)
    iota = lax.iota(jnp.int32, L)

    copies = []
    for i in range(NUM_FIELDS):
        col = jnp.full((L,), i, jnp.int32)

        def trans(j, _, i=i, col=col):
            rows = j * L + iota
            xts[i][pl.ds(j * L, L)] = (
                plsc.load_gather(xv2, [rows, col]) + i * VOCAB
            )
            return 0

        lax.fori_loop(0, NCOL, trans, 0)
        copies.append(
            pltpu.async_copy(
                tab_ref.at[xts[i]], rowsv.at[pl.ds(i * BPW, BPW)], sem
            )
        )
    for c in copies:
        c.wait()

    wvs = [
        plsc.load_gather(parv, [jnp.full((L,), i, jnp.int32)])
        for i in range(NUM_FIELDS)
    ]
    bconst = plsc.load_gather(parv, [jnp.full((L,), NUM_FIELDS, jnp.int32)])

    def col(j, _):
        b0 = j * L
        acc = bconst
        for i in range(NUM_FIELDS):
            acc = acc + wvs[i] * rowsv[pl.ds(i * BPW + b0, L)]
        outv[pl.ds(b0, L)] = 1.0 / (1.0 + jnp.exp(-acc))
        return 0

    lax.fori_loop(0, NCOL, col, 0)

    pltpu.sync_copy(outv, out_ref.at[pl.ds(base_b, BPW)])


@jax.jit
def _run(x, tables, W, b, bias):
    params = jnp.concatenate(
        [
            W.reshape(-1).astype(jnp.float32),
            (b + bias).reshape(-1).astype(jnp.float32),
            jnp.zeros(128 - NUM_FIELDS - 1, jnp.float32),
        ]
    )
    x = x.astype(jnp.int32)
    mesh = plsc.VectorSubcoreMesh(core_axis_name="c", subcore_axis_name="s")

    stage = pl.kernel(
        _stage_body,
        out_type=jax.ShapeDtypeStruct((NUM_FIELDS * VOCAB,), jnp.float32),
        mesh=mesh,
        compiler_params=pltpu.CompilerParams(needs_layout_passes=False),
        scratch_types=[
            pltpu.VMEM((CHW,), jnp.float32),
            pltpu.VMEM((CHW,), jnp.float32),
            pltpu.VMEM((TAIL,), jnp.float32),
            pltpu.SemaphoreType.DMA,
            pltpu.SemaphoreType.DMA,
        ],
    )
    flat = stage(tables)

    gather = pl.kernel(
        _gather_body,
        out_type=jax.ShapeDtypeStruct((BATCH,), jnp.float32),
        mesh=mesh,
        compiler_params=pltpu.CompilerParams(needs_layout_passes=False),
        scratch_types=[
            pltpu.VMEM((BPW, NUM_FIELDS), jnp.int32),
            pltpu.VMEM((NUM_FIELDS * BPW,), jnp.float32),
            pltpu.VMEM((BPW,), jnp.float32),
            pltpu.VMEM((128,), jnp.float32),
            pltpu.SemaphoreType.DMA,
        ]
        + [pltpu.VMEM((BPW,), jnp.int32) for _ in range(NUM_FIELDS)],
    )
    return gather(x, flat, params).reshape(BATCH, 1)


def kernel(x, tables, W, b, bias):
    return _run(x, tables, W, b, bias)


# speedup vs baseline: 1.1442x; 1.0341x over previous
import functools

import jax
import jax.numpy as jnp
from jax import lax
from jax.experimental import pallas as pl
from jax.experimental.pallas import tpu as pltpu
from jax.experimental.pallas import tpu_sc as plsc

NUM_FIELDS = 26
VOCAB = 1000000
BATCH = 16384

NC = 2
NS = 16
L = 16
NW = NC * NS
BPW = BATCH // NW
NCOL = BPW // L

CHW = 31232
TAIL = VOCAB - NW * CHW


NBUF = 3


def _stage_body(tab_ref, flat_ref, buf0, buf1, buf2, tbuf, semi, semo):
    wid = lax.axis_index("s") * NC + lax.axis_index("c")
    src_off = pl.multiple_of(wid * CHW, 128)
    bufs = [buf0, buf1, buf2]

    def cin(r):
        return pltpu.async_copy(
            tab_ref.at[r, pl.ds(src_off, CHW)], bufs[r % NBUF], semi
        )

    def cout(r):
        dst_off = pl.multiple_of(r * VOCAB + wid * CHW, 8)
        return pltpu.async_copy(
            bufs[r % NBUF], flat_ref.at[pl.ds(dst_off, CHW)], semo
        )

    ins = [cin(r) for r in range(NBUF - 1)]
    outs = []
    for r in range(NUM_FIELDS):
        ins[r].wait()
        if r >= 1:
            outs[r - 1].wait()
        if r + NBUF - 1 < NUM_FIELDS:
            ins.append(cin(r + NBUF - 1))
        outs.append(cout(r))
    outs[NUM_FIELDS - 1].wait()

    for r in range(NUM_FIELDS):

        @pl.when(wid == r)
        def _():
            pltpu.sync_copy(tab_ref.at[r, pl.ds(NW * CHW, TAIL)], tbuf)
            pltpu.sync_copy(tbuf, flat_ref.at[pl.ds(r * VOCAB + NW * CHW, TAIL)])


def _gather_body(x_ref, tab_ref, par_ref, out_ref, xv2, rowsv, outv, parv,
                 sem, *xts):
    wid = lax.axis_index("s") * NC + lax.axis_index("c")
    base_b = wid * BPW

    pltpu.sync_copy(x_ref.at[pl.ds(base_b, BPW), :], xv2)
    pltpu.sync_copy(par_ref, parv)
    iota = lax.iota(jnp.int32, L)

    copies = []
    for i in range(NUM_FIELDS):
        col = jnp.full((L,), i, jnp.int32)

        def trans(j, _, i=i, col=col):
            rows = j * L + iota
            xts[i][pl.ds(j * L, L)] = (
                plsc.load_gather(xv2, [rows, col]) + i * VOCAB
            )
            return 0

        lax.fori_loop(0, NCOL, trans, 0)
        copies.append(
            pltpu.async_copy(
                tab_ref.at[xts[i]], rowsv.at[pl.ds(i * BPW, BPW)], sem
            )
        )
    for c in copies:
        c.wait()

    wvs = [
        plsc.load_gather(parv, [jnp.full((L,), i, jnp.int32)])
        for i in range(NUM_FIELDS)
    ]
    bconst = plsc.load_gather(parv, [jnp.full((L,), NUM_FIELDS, jnp.int32)])

    def col(j, _):
        b0 = j * L
        acc = bconst
        for i in range(NUM_FIELDS):
            acc = acc + wvs[i] * rowsv[pl.ds(i * BPW + b0, L)]
        outv[pl.ds(b0, L)] = 1.0 / (1.0 + jnp.exp(-acc))
        return 0

    lax.fori_loop(0, NCOL, col, 0)

    pltpu.sync_copy(outv, out_ref.at[pl.ds(base_b, BPW)])


@jax.jit
def _run(x, tables, W, b, bias):
    params = jnp.concatenate(
        [
            W.reshape(-1).astype(jnp.float32),
            (b + bias).reshape(-1).astype(jnp.float32),
            jnp.zeros(128 - NUM_FIELDS - 1, jnp.float32),
        ]
    )
    x = x.astype(jnp.int32)
    mesh = plsc.VectorSubcoreMesh(core_axis_name="c", subcore_axis_name="s")

    stage = pl.kernel(
        _stage_body,
        out_type=jax.ShapeDtypeStruct((NUM_FIELDS * VOCAB,), jnp.float32),
        mesh=mesh,
        compiler_params=pltpu.CompilerParams(needs_layout_passes=False),
        scratch_types=[
            pltpu.VMEM((CHW,), jnp.float32),
            pltpu.VMEM((CHW,), jnp.float32),
            pltpu.VMEM((CHW,), jnp.float32),
            pltpu.VMEM((TAIL,), jnp.float32),
            pltpu.SemaphoreType.DMA,
            pltpu.SemaphoreType.DMA,
        ],
    )
    flat = stage(tables)

    gather = pl.kernel(
        _gather_body,
        out_type=jax.ShapeDtypeStruct((BATCH,), jnp.float32),
        mesh=mesh,
        compiler_params=pltpu.CompilerParams(needs_layout_passes=False),
        scratch_types=[
            pltpu.VMEM((BPW, NUM_FIELDS), jnp.int32),
            pltpu.VMEM((NUM_FIELDS * BPW,), jnp.float32),
            pltpu.VMEM((BPW,), jnp.float32),
            pltpu.VMEM((128,), jnp.float32),
            pltpu.SemaphoreType.DMA,
        ]
        + [pltpu.VMEM((BPW,), jnp.int32) for _ in range(NUM_FIELDS)],
    )
    return gather(x, flat, params).reshape(BATCH, 1)


def kernel(x, tables, W, b, bias):
    return _run(x, tables, W, b, bias)
